# baseline (speedup 1.0000x reference)
import functools

import jax
import jax.numpy as jnp
from jax import lax
from jax.experimental import pallas as pl
from jax.experimental.pallas import tpu as pltpu
from jax.experimental.pallas import tpu_sc as plsc

N = 10000
E = 160000
EROWS = 1280
NPAD = 10240
DUMP = 10016
NC = 2
NS = 16


def _mesh():
  return plsc.VectorSubcoreMesh(
      core_axis_name="c", subcore_axis_name="s", num_cores=NC, num_subcores=NS
  )


def _deg_body(dst_hbm, parts_hbm, acc, dstall, ones, zrow, sem):
  c = lax.axis_index("c")
  s = lax.axis_index("s")

  for i in range(40):
    zrow[pl.ds(i * 16, 16)] = jnp.zeros((16,), jnp.float32)
  for i in range(8):
    ones[pl.ds(i * 16, 16)] = jnp.ones((16,), jnp.float32)
  pltpu.sync_copy(zrow, acc.at[pl.ds(s * 640, 640)])
  pltpu.sync_copy(dst_hbm.at[pl.ds(c * 640 + s * 40, 40)], dstall)
  plsc.subcore_barrier()

  def loop_body(k, carry):
    pltpu.async_copy(ones, acc.at[dstall.at[k]], sem, add=True)
    return carry

  lax.fori_loop(0, 40, loop_body, 0)

  def drain_body(k, carry):
    pltpu.make_async_copy(ones, acc.at[dstall.at[k]], sem).wait()
    return carry

  lax.fori_loop(0, 40, drain_body, 0)

  plsc.subcore_barrier()
  pltpu.sync_copy(acc.at[pl.ds(s * 640, 640)], parts_hbm.at[c, pl.ds(s * 640, 640)])


def _sc_degree(dst2):
  return pl.kernel(
      _deg_body,
      out_type=jax.ShapeDtypeStruct((NC, NPAD), jnp.float32),
      mesh=_mesh(),
      scratch_types=[
          pltpu.VMEM_SHARED((NPAD,), jnp.float32),
          pltpu.VMEM((40, 128), jnp.int32),
          pltpu.VMEM((128,), jnp.float32),
          pltpu.VMEM((640,), jnp.float32),
          pltpu.SemaphoreType.DMA,
      ],
  )(dst2)


def _prop_body(g_hbm, src_hbm, dst_hbm, out_hbm, acc, srcall, dstall, rows0,
               rows1, gsem, psem, *, chsplit):
  c = lax.axis_index("c")
  s = lax.axis_index("s")
  ch0 = c * 128

  def zero_row(i, carry):
    for j in range(8):
      rows0[i, pl.ds(j * 16, 16)] = jnp.zeros((16,), jnp.float32)
    return carry

  lax.fori_loop(0, 128, zero_row, 0)
  for b in range(5):
    pltpu.sync_copy(rows0, acc.at[pl.ds(s * 640 + b * 128, 128)])

  if chsplit:
    nphase, base0 = 2, s * 80
  else:
    nphase, base0 = 1, c * 640 + s * 40

  def gsrc(k):
    if chsplit:
      return g_hbm.at[srcall.at[k], pl.ds(ch0, 128)]
    return g_hbm.at[srcall.at[k]]

  def gather(k, buf):
    pltpu.async_copy(gsrc(k), buf, gsem)

  def gwait(k, buf):
    pltpu.make_async_copy(gsrc(k), buf, gsem).wait()

  def scat(k, buf):
    pltpu.async_copy(buf, acc.at[dstall.at[k]], psem, add=True)

  def swait(k, buf):
    pltpu.make_async_copy(buf, acc.at[dstall.at[k]], psem).wait()

  first = True
  for phase in range(nphase):
    base = base0 + phase * 40
    pltpu.sync_copy(src_hbm.at[pl.ds(base, 40)], srcall)
    pltpu.sync_copy(dst_hbm.at[pl.ds(base, 40)], dstall)
    if first:
      plsc.subcore_barrier()
      first = False

    gather(0, rows0)

    def pair_body(p, carry):
      a = 2 * p
      gather(a + 1, rows1)
      gwait(a, rows0)
      scat(a, rows0)
      gwait(a + 1, rows1)
      scat(a + 1, rows1)
      swait(a, rows0)
      @pl.when(p < 19)
      def _():
        gather(a + 2, rows0)
      swait(a + 1, rows1)
      return carry

    lax.fori_loop(0, 20, pair_body, 0)

  plsc.subcore_barrier()
  for b in range(5):
    pltpu.sync_copy(
        acc.at[pl.ds(s * 640 + b * 128, 128)],
        out_hbm.at[c, pl.ds(s * 640 + b * 128, 128)],
    )


def _sc_propagate(g, src2, dst2, chsplit):
  body = functools.partial(_prop_body, chsplit=chsplit)
  return pl.kernel(
      body,
      out_type=jax.ShapeDtypeStruct((NC, NPAD, 128), jnp.float32),
      mesh=_mesh(),
      scratch_types=[
          pltpu.VMEM_SHARED((NPAD, 128), jnp.float32),
          pltpu.VMEM((40, 128), jnp.int32),
          pltpu.VMEM((40, 128), jnp.int32),
          pltpu.VMEM((128, 128), jnp.float32),
          pltpu.VMEM((128, 128), jnp.float32),
          pltpu.SemaphoreType.DMA,
          pltpu.SemaphoreType.DMA,
      ],
  )(g, src2, dst2)


_RB = 1000


def _dis(d0, d1):
  deg = d0 + d1 + 1.0
  return lax.rsqrt(deg)


def _g1_body(d0_ref, d1_ref, x_ref, g1_ref):
  dis = _dis(d0_ref[...], d1_ref[...])
  g1_ref[...] = x_ref[...] * dis


def _tc_g1(d0, d1, x):
  return pl.pallas_call(
      _g1_body,
      grid=(N // _RB,),
      in_specs=[
          pl.BlockSpec((_RB, 1), lambda i: (i, 0)),
          pl.BlockSpec((_RB, 1), lambda i: (i, 0)),
          pl.BlockSpec((_RB, 256), lambda i: (i, 0)),
      ],
      out_specs=pl.BlockSpec((_RB, 256), lambda i: (i, 0)),
      out_shape=jax.ShapeDtypeStruct((N, 256), jnp.float32),
  )(d0, d1, x)


def _mm_body(d0_ref, d1_ref, p1a_ref, p1b_ref, x_ref, w1_ref, b1_ref, w2_ref,
             z_ref, g2_ref):
  dis = _dis(d0_ref[...], d1_ref[...])
  dis2 = dis * dis
  p1 = jnp.concatenate([p1a_ref[0], p1b_ref[0]], axis=1)
  ax = p1 * dis + x_ref[...] * dis2
  h = jnp.maximum(
      jnp.dot(ax, w1_ref[...], preferred_element_type=jnp.float32)
      + b1_ref[...],
      0.0,
  )
  z = jnp.dot(h, w2_ref[...], preferred_element_type=jnp.float32)
  z_ref[...] = z
  g2_ref[...] = z * dis


def _tc_mm(d0, d1, p1a, p1b, x, w1, b1r, w2):
  return pl.pallas_call(
      _mm_body,
      grid=(N // _RB,),
      in_specs=[
          pl.BlockSpec((_RB, 1), lambda i: (i, 0)),
          pl.BlockSpec((_RB, 1), lambda i: (i, 0)),
          pl.BlockSpec((1, _RB, 128), lambda i: (0, i, 0)),
          pl.BlockSpec((1, _RB, 128), lambda i: (1, i, 0)),
          pl.BlockSpec((_RB, 256), lambda i: (i, 0)),
          pl.BlockSpec((256, 512), lambda i: (0, 0)),
          pl.BlockSpec((1, 512), lambda i: (0, 0)),
          pl.BlockSpec((512, 128), lambda i: (0, 0)),
      ],
      out_specs=[
          pl.BlockSpec((_RB, 128), lambda i: (i, 0)),
          pl.BlockSpec((_RB, 128), lambda i: (i, 0)),
      ],
      out_shape=[
          jax.ShapeDtypeStruct((N, 128), jnp.float32),
          jax.ShapeDtypeStruct((N, 128), jnp.float32),
      ],
  )(d0, d1, p1a, p1b, x, w1, b1r, w2)


def _out_body(d0_ref, d1_ref, p2a_ref, p2b_ref, z_ref, b2_ref, o_ref):
  dis = _dis(d0_ref[...], d1_ref[...])
  dis2 = dis * dis
  p2 = p2a_ref[0] + p2b_ref[0]
  u = p2 * dis + z_ref[...] * dis2 + b2_ref[...]
  m = jnp.max(u, axis=1, keepdims=True)
  t = u - m
  lse = jnp.log(jnp.sum(jnp.exp(t), axis=1, keepdims=True))
  o_ref[...] = t - lse


def _tc_out(d0, d1, p2a, p2b, z, b2r):
  return pl.pallas_call(
      _out_body,
      grid=(N // _RB,),
      in_specs=[
          pl.BlockSpec((_RB, 1), lambda i: (i, 0)),
          pl.BlockSpec((_RB, 1), lambda i: (i, 0)),
          pl.BlockSpec((1, _RB, 128), lambda i: (0, i, 0)),
          pl.BlockSpec((1, _RB, 128), lambda i: (1, i, 0)),
          pl.BlockSpec((_RB, 128), lambda i: (i, 0)),
          pl.BlockSpec((1, 128), lambda i: (0, 0)),
      ],
      out_specs=pl.BlockSpec((_RB, 128), lambda i: (i, 0)),
      out_shape=jax.ShapeDtypeStruct((N, 128), jnp.float32),
  )(d0, d1, p2a, p2b, z, b2r)


def kernel(x, edge_index, W1, b1, W2, b2):
  npe = EROWS * 128 - E
  ar = jnp.arange(npe, dtype=jnp.int32)
  src2 = jnp.concatenate([edge_index[0], (ar * 61) % N]).reshape(EROWS, 128)
  dst2 = jnp.concatenate([edge_index[1], DUMP + (ar % 128)]).reshape(EROWS, 128)

  parts = _sc_degree(dst2)
  d0 = parts[0, :N].reshape(N, 1)
  d1 = parts[1, :N].reshape(N, 1)

  g1 = _tc_g1(d0, d1, x)
  p1 = _sc_propagate(g1, src2, dst2, True)
  z, g2 = _tc_mm(d0, d1, p1, p1, x, W1,
                 b1.reshape(1, 512), W2)
  p2 = _sc_propagate(g2, src2, dst2, False)
  return _tc_out(d0, d1, p2, p2, z, b2.reshape(1, 128))


# speedup vs baseline: 1.0489x; 1.0489x over previous
import functools

import jax
import jax.numpy as jnp
from jax import lax
from jax.experimental import pallas as pl
from jax.experimental.pallas import tpu as pltpu
from jax.experimental.pallas import tpu_sc as plsc

N = 10000
E = 160000
EROWS = 1280
NPAD = 10240
DUMP = 10016
NC = 2
NS = 16


def _mesh():
  return plsc.VectorSubcoreMesh(
      core_axis_name="c", subcore_axis_name="s", num_cores=NC, num_subcores=NS
  )


def _deg_body(e3_hbm, parts_hbm, src2_hbm, dst2_hbm, acc, srcall, dstall,
              ones, zrow, sem):
  c = lax.axis_index("c")
  s = lax.axis_index("s")

  for i in range(40):
    zrow[pl.ds(i * 16, 16)] = jnp.zeros((16,), jnp.float32)
  for i in range(8):
    ones[pl.ds(i * 16, 16)] = jnp.ones((16,), jnp.float32)
  pltpu.sync_copy(zrow, acc.at[pl.ds(s * 640, 640)])

  rbase = c * 625 + s * 39
  for k in range(39):
    pltpu.async_copy(e3_hbm.at[0, rbase + k], srcall.at[k], sem)
    pltpu.async_copy(e3_hbm.at[1, rbase + k], dstall.at[k], sem)
  for k in range(39):
    pltpu.make_async_copy(e3_hbm.at[0, rbase + k], srcall.at[k], sem).wait()
    pltpu.make_async_copy(e3_hbm.at[1, rbase + k], dstall.at[k], sem).wait()
  lanes = lax.iota(jnp.int32, 16)
  @pl.when(s == 0)
  def _():
    pltpu.sync_copy(e3_hbm.at[0, c * 625 + 624], srcall.at[39])
    pltpu.sync_copy(e3_hbm.at[1, c * 625 + 624], dstall.at[39])
  @pl.when(s != 0)
  def _():
    for j in range(8):
      srcall[39, pl.ds(j * 16, 16)] = lanes + j * 16
      dstall[39, pl.ds(j * 16, 16)] = lanes + (j * 16 + DUMP)
  plsc.subcore_barrier()

  def loop_body(k, carry):
    pltpu.async_copy(ones, acc.at[dstall.at[k]], sem, add=True)
    return carry

  lax.fori_loop(0, 40, loop_body, 0)

  def drain_body(k, carry):
    pltpu.make_async_copy(ones, acc.at[dstall.at[k]], sem).wait()
    return carry

  lax.fori_loop(0, 40, drain_body, 0)

  obase = c * 640 + s * 40
  pltpu.sync_copy(srcall, src2_hbm.at[pl.ds(obase, 40)])
  pltpu.sync_copy(dstall, dst2_hbm.at[pl.ds(obase, 40)])

  plsc.subcore_barrier()
  pltpu.sync_copy(acc.at[pl.ds(s * 640, 640)], parts_hbm.at[c, pl.ds(s * 640, 640)])


def _sc_degree(e3):
  return pl.kernel(
      _deg_body,
      out_type=[
          jax.ShapeDtypeStruct((NC, NPAD), jnp.float32),
          jax.ShapeDtypeStruct((EROWS, 128), jnp.int32),
          jax.ShapeDtypeStruct((EROWS, 128), jnp.int32),
      ],
      mesh=_mesh(),
      scratch_types=[
          pltpu.VMEM_SHARED((NPAD,), jnp.float32),
          pltpu.VMEM((40, 128), jnp.int32),
          pltpu.VMEM((40, 128), jnp.int32),
          pltpu.VMEM((128,), jnp.float32),
          pltpu.VMEM((640,), jnp.float32),
          pltpu.SemaphoreType.DMA,
      ],
  )(e3)


def _prop_body(g_hbm, src_hbm, dst_hbm, out_hbm, acc, srcall, dstall, rows0,
               rows1, gsem, psem, *, chsplit):
  c = lax.axis_index("c")
  s = lax.axis_index("s")
  ch0 = c * 128

  def zero_row(i, carry):
    for j in range(8):
      rows0[i, pl.ds(j * 16, 16)] = jnp.zeros((16,), jnp.float32)
    return carry

  lax.fori_loop(0, 128, zero_row, 0)
  for b in range(5):
    pltpu.sync_copy(rows0, acc.at[pl.ds(s * 640 + b * 128, 128)])

  if chsplit:
    nphase, base0 = 2, s * 80
  else:
    nphase, base0 = 1, c * 640 + s * 40

  def gsrc(k):
    if chsplit:
      return g_hbm.at[srcall.at[k], pl.ds(ch0, 128)]
    return g_hbm.at[srcall.at[k]]

  def gather(k, buf):
    pltpu.async_copy(gsrc(k), buf, gsem)

  def gwait(k, buf):
    pltpu.make_async_copy(gsrc(k), buf, gsem).wait()

  def scat(k, buf):
    pltpu.async_copy(buf, acc.at[dstall.at[k]], psem, add=True)

  def swait(k, buf):
    pltpu.make_async_copy(buf, acc.at[dstall.at[k]], psem).wait()

  first = True
  for phase in range(nphase):
    base = base0 + phase * 40
    pltpu.sync_copy(src_hbm.at[pl.ds(base, 40)], srcall)
    pltpu.sync_copy(dst_hbm.at[pl.ds(base, 40)], dstall)
    if first:
      plsc.subcore_barrier()
      first = False

    gather(0, rows0)

    def pair_body(p, carry):
      a = 2 * p
      gather(a + 1, rows1)
      gwait(a, rows0)
      scat(a, rows0)
      swait(a, rows0)
      @pl.when(p < 19)
      def _():
        gather(a + 2, rows0)
      gwait(a + 1, rows1)
      scat(a + 1, rows1)
      swait(a + 1, rows1)
      return carry

    lax.fori_loop(0, 20, pair_body, 0)

  plsc.subcore_barrier()
  for b in range(5):
    pltpu.sync_copy(
        acc.at[pl.ds(s * 640 + b * 128, 128)],
        out_hbm.at[c, pl.ds(s * 640 + b * 128, 128)],
    )


def _sc_propagate(g, src2, dst2, chsplit):
  body = functools.partial(_prop_body, chsplit=chsplit)
  return pl.kernel(
      body,
      out_type=jax.ShapeDtypeStruct((NC, NPAD, 128), jnp.float32),
      mesh=_mesh(),
      scratch_types=[
          pltpu.VMEM_SHARED((NPAD, 128), jnp.float32),
          pltpu.VMEM((40, 128), jnp.int32),
          pltpu.VMEM((40, 128), jnp.int32),
          pltpu.VMEM((128, 128), jnp.float32),
          pltpu.VMEM((128, 128), jnp.float32),
          pltpu.SemaphoreType.DMA,
          pltpu.SemaphoreType.DMA,
      ],
  )(g, src2, dst2)


_RB = 1000


def _dis(d0, d1):
  deg = d0 + d1 + 1.0
  return lax.rsqrt(deg)


def _g1_body(d0_ref, d1_ref, x_ref, g1_ref):
  dis = _dis(d0_ref[...], d1_ref[...])
  g1_ref[...] = x_ref[...] * dis


def _tc_g1(d0, d1, x):
  return pl.pallas_call(
      _g1_body,
      grid=(N // _RB,),
      in_specs=[
          pl.BlockSpec((_RB, 1), lambda i: (i, 0)),
          pl.BlockSpec((_RB, 1), lambda i: (i, 0)),
          pl.BlockSpec((_RB, 256), lambda i: (i, 0)),
      ],
      out_specs=pl.BlockSpec((_RB, 256), lambda i: (i, 0)),
      out_shape=jax.ShapeDtypeStruct((N, 256), jnp.float32),
  )(d0, d1, x)


def _mm_body(d0_ref, d1_ref, p1a_ref, p1b_ref, x_ref, w1_ref, b1_ref, w2_ref,
             z_ref, g2_ref):
  dis = _dis(d0_ref[...], d1_ref[...])
  dis2 = dis * dis
  p1 = jnp.concatenate([p1a_ref[0], p1b_ref[0]], axis=1)
  ax = p1 * dis + x_ref[...] * dis2
  h = jnp.maximum(
      jnp.dot(ax, w1_ref[...], preferred_element_type=jnp.float32)
      + b1_ref[...],
      0.0,
  )
  z = jnp.dot(h, w2_ref[...], preferred_element_type=jnp.float32)
  z_ref[...] = z
  g2_ref[...] = z * dis


def _tc_mm(d0, d1, p1a, p1b, x, w1, b1r, w2):
  return pl.pallas_call(
      _mm_body,
      grid=(N // _RB,),
      in_specs=[
          pl.BlockSpec((_RB, 1), lambda i: (i, 0)),
          pl.BlockSpec((_RB, 1), lambda i: (i, 0)),
          pl.BlockSpec((1, _RB, 128), lambda i: (0, i, 0)),
          pl.BlockSpec((1, _RB, 128), lambda i: (1, i, 0)),
          pl.BlockSpec((_RB, 256), lambda i: (i, 0)),
          pl.BlockSpec((256, 512), lambda i: (0, 0)),
          pl.BlockSpec((1, 512), lambda i: (0, 0)),
          pl.BlockSpec((512, 128), lambda i: (0, 0)),
      ],
      out_specs=[
          pl.BlockSpec((_RB, 128), lambda i: (i, 0)),
          pl.BlockSpec((_RB, 128), lambda i: (i, 0)),
      ],
      out_shape=[
          jax.ShapeDtypeStruct((N, 128), jnp.float32),
          jax.ShapeDtypeStruct((N, 128), jnp.float32),
      ],
  )(d0, d1, p1a, p1b, x, w1, b1r, w2)


def _out_body(d0_ref, d1_ref, p2a_ref, p2b_ref, z_ref, b2_ref, o_ref):
  dis = _dis(d0_ref[...], d1_ref[...])
  dis2 = dis * dis
  p2 = p2a_ref[0] + p2b_ref[0]
  u = p2 * dis + z_ref[...] * dis2 + b2_ref[...]
  m = jnp.max(u, axis=1, keepdims=True)
  t = u - m
  lse = jnp.log(jnp.sum(jnp.exp(t), axis=1, keepdims=True))
  o_ref[...] = t - lse


def _tc_out(d0, d1, p2a, p2b, z, b2r):
  return pl.pallas_call(
      _out_body,
      grid=(N // _RB,),
      in_specs=[
          pl.BlockSpec((_RB, 1), lambda i: (i, 0)),
          pl.BlockSpec((_RB, 1), lambda i: (i, 0)),
          pl.BlockSpec((1, _RB, 128), lambda i: (0, i, 0)),
          pl.BlockSpec((1, _RB, 128), lambda i: (1, i, 0)),
          pl.BlockSpec((_RB, 128), lambda i: (i, 0)),
          pl.BlockSpec((1, 128), lambda i: (0, 0)),
      ],
      out_specs=pl.BlockSpec((_RB, 128), lambda i: (i, 0)),
      out_shape=jax.ShapeDtypeStruct((N, 128), jnp.float32),
  )(d0, d1, p2a, p2b, z, b2r)


def kernel(x, edge_index, W1, b1, W2, b2):
  e3 = edge_index.reshape(2, E // 128, 128)
  parts, src2, dst2 = _sc_degree(e3)
  d0 = parts[0, :N].reshape(N, 1)
  d1 = parts[1, :N].reshape(N, 1)

  g1 = _tc_g1(d0, d1, x)
  p1 = _sc_propagate(g1, src2, dst2, True)
  z, g2 = _tc_mm(d0, d1, p1, p1, x, W1,
                 b1.reshape(1, 512), W2)
  p2 = _sc_propagate(g2, src2, dst2, False)
  return _tc_out(d0, d1, p2, p2, z, b2.reshape(1, 128))


# speedup vs baseline: 1.0695x; 1.0197x over previous
import functools

import jax
import jax.numpy as jnp
from jax import lax
from jax.experimental import pallas as pl
from jax.experimental.pallas import tpu as pltpu
from jax.experimental.pallas import tpu_sc as plsc

N = 10000
E = 160000
EROWS = 1280
NPAD = 10240
DUMP = 10016
NC = 2
NS = 16


def _mesh():
  return plsc.VectorSubcoreMesh(
      core_axis_name="c", subcore_axis_name="s", num_cores=NC, num_subcores=NS
  )


def _deg_body(e3_hbm, parts_hbm, src2_hbm, dst2_hbm, acc, srcall, dstall,
              ones, zrow, sem):
  c = lax.axis_index("c")
  s = lax.axis_index("s")

  for i in range(40):
    zrow[pl.ds(i * 16, 16)] = jnp.zeros((16,), jnp.float32)
  for i in range(8):
    ones[pl.ds(i * 16, 16)] = jnp.ones((16,), jnp.float32)
  pltpu.sync_copy(zrow, acc.at[pl.ds(s * 640, 640)])

  rbase = c * 625 + s * 39
  for k in range(39):
    pltpu.async_copy(e3_hbm.at[0, rbase + k], srcall.at[k], sem)
    pltpu.async_copy(e3_hbm.at[1, rbase + k], dstall.at[k], sem)
  for k in range(39):
    pltpu.make_async_copy(e3_hbm.at[0, rbase + k], srcall.at[k], sem).wait()
    pltpu.make_async_copy(e3_hbm.at[1, rbase + k], dstall.at[k], sem).wait()
  lanes = lax.iota(jnp.int32, 16)
  @pl.when(s == 0)
  def _():
    pltpu.sync_copy(e3_hbm.at[0, c * 625 + 624], srcall.at[39])
    pltpu.sync_copy(e3_hbm.at[1, c * 625 + 624], dstall.at[39])
  @pl.when(s != 0)
  def _():
    for j in range(8):
      srcall[39, pl.ds(j * 16, 16)] = lanes + j * 16
      dstall[39, pl.ds(j * 16, 16)] = lanes + (j * 16 + DUMP)
  plsc.subcore_barrier()

  def loop_body(k, carry):
    pltpu.async_copy(ones, acc.at[dstall.at[k]], sem, add=True)
    return carry

  lax.fori_loop(0, 40, loop_body, 0)

  def drain_body(k, carry):
    pltpu.make_async_copy(ones, acc.at[dstall.at[k]], sem).wait()
    return carry

  lax.fori_loop(0, 40, drain_body, 0)

  obase = c * 640 + s * 40
  pltpu.sync_copy(srcall, src2_hbm.at[pl.ds(obase, 40)])
  pltpu.sync_copy(dstall, dst2_hbm.at[pl.ds(obase, 40)])

  plsc.subcore_barrier()
  pltpu.sync_copy(acc.at[pl.ds(s * 640, 640)],
                  parts_hbm.at[c, pl.ds(s * 640, 640)])


def _sc_degree(e3):
  return pl.kernel(
      _deg_body,
      out_type=[
          jax.ShapeDtypeStruct((NC, NPAD), jnp.float32),
          jax.ShapeDtypeStruct((EROWS, 128), jnp.int32),
          jax.ShapeDtypeStruct((EROWS, 128), jnp.int32),
      ],
      mesh=_mesh(),
      scratch_types=[
          pltpu.VMEM_SHARED((NPAD,), jnp.float32),
          pltpu.VMEM((40, 128), jnp.int32),
          pltpu.VMEM((40, 128), jnp.int32),
          pltpu.VMEM((128,), jnp.float32),
          pltpu.VMEM((640,), jnp.float32),
          pltpu.SemaphoreType.DMA,
      ],
  )(e3)


def _prop_body(g_hbm, src_hbm, dst_hbm, out_hbm, acc, srcall, dstall, rows0,
               rows1, gsem, psem, *, chsplit):
  c = lax.axis_index("c")
  s = lax.axis_index("s")
  ch0 = c * 128

  def zero_row(i, carry):
    for j in range(8):
      rows0[i, pl.ds(j * 16, 16)] = jnp.zeros((16,), jnp.float32)
    return carry

  lax.fori_loop(0, 128, zero_row, 0)
  for b in range(5):
    pltpu.sync_copy(rows0, acc.at[pl.ds(s * 640 + b * 128, 128)])

  if chsplit:
    nphase, base0 = 2, s * 80
  else:
    nphase, base0 = 1, c * 640 + s * 40

  def gsrc(k):
    if chsplit:
      return g_hbm.at[srcall.at[k], pl.ds(ch0, 128)]
    return g_hbm.at[srcall.at[k]]

  def gather(k, buf):
    pltpu.async_copy(gsrc(k), buf, gsem)

  def gwait(k, buf):
    pltpu.make_async_copy(gsrc(k), buf, gsem).wait()

  def scat(k, buf):
    pltpu.async_copy(buf, acc.at[dstall.at[k]], psem, add=True)

  def swait(k, buf):
    pltpu.make_async_copy(buf, acc.at[dstall.at[k]], psem).wait()

  first = True
  for phase in range(nphase):
    base = base0 + phase * 40
    pltpu.sync_copy(src_hbm.at[pl.ds(base, 40)], srcall)
    pltpu.sync_copy(dst_hbm.at[pl.ds(base, 40)], dstall)
    if first:
      plsc.subcore_barrier()
      first = False

    gather(0, rows0)

    def pair_body(p, carry):
      a = 2 * p
      gather(a + 1, rows1)
      gwait(a, rows0)
      scat(a, rows0)
      swait(a, rows0)
      @pl.when(p < 19)
      def _():
        gather(a + 2, rows0)
      gwait(a + 1, rows1)
      scat(a + 1, rows1)
      swait(a + 1, rows1)
      return carry

    lax.fori_loop(0, 20, pair_body, 0)

  plsc.subcore_barrier()
  for b in range(5):
    pltpu.sync_copy(
        acc.at[pl.ds(s * 640 + b * 128, 128)],
        out_hbm.at[c, pl.ds(s * 640 + b * 128, 128)],
    )


def _sc_propagate(g, src2, dst2, chsplit):
  body = functools.partial(_prop_body, chsplit=chsplit)
  return pl.kernel(
      body,
      out_type=jax.ShapeDtypeStruct((NC, NPAD, 128), jnp.float32),
      mesh=_mesh(),
      scratch_types=[
          pltpu.VMEM_SHARED((NPAD, 128), jnp.float32),
          pltpu.VMEM((40, 128), jnp.int32),
          pltpu.VMEM((40, 128), jnp.int32),
          pltpu.VMEM((128, 128), jnp.float32),
          pltpu.VMEM((128, 128), jnp.float32),
          pltpu.SemaphoreType.DMA,
          pltpu.SemaphoreType.DMA,
      ],
  )(g, src2, dst2)


_RB = 2000


def _dis(d0, d1):
  deg = d0 + d1 + 1.0
  return lax.rsqrt(deg)


def _g1_body(d0_ref, d1_ref, x_ref, g1_ref):
  dis = _dis(d0_ref[...], d1_ref[...])
  g1_ref[...] = x_ref[...] * dis


def _tc_g1(d0, d1, x):
  return pl.pallas_call(
      _g1_body,
      grid=(N // _RB,),
      in_specs=[
          pl.BlockSpec((_RB, 1), lambda i: (i, 0)),
          pl.BlockSpec((_RB, 1), lambda i: (i, 0)),
          pl.BlockSpec((_RB, 256), lambda i: (i, 0)),
      ],
      out_specs=pl.BlockSpec((_RB, 256), lambda i: (i, 0)),
      out_shape=jax.ShapeDtypeStruct((N, 256), jnp.float32),
  )(d0, d1, x)


def _mm_body(d0_ref, d1_ref, p1a_ref, p1b_ref, x_ref, w1_ref, b1_ref, w2_ref,
             z_ref, g2_ref):
  dis = _dis(d0_ref[...], d1_ref[...])
  dis2 = dis * dis
  p1 = jnp.concatenate([p1a_ref[0], p1b_ref[0]], axis=1)
  ax = p1 * dis + x_ref[...] * dis2
  h = jnp.maximum(
      jnp.dot(ax, w1_ref[...], preferred_element_type=jnp.float32)
      + b1_ref[...],
      0.0,
  )
  z = jnp.dot(h, w2_ref[...], preferred_element_type=jnp.float32)
  z_ref[...] = z
  g2_ref[...] = z * dis


def _tc_mm(d0, d1, p1, x, w1, b1r, w2):
  return pl.pallas_call(
      _mm_body,
      grid=(N // _RB,),
      in_specs=[
          pl.BlockSpec((_RB, 1), lambda i: (i, 0)),
          pl.BlockSpec((_RB, 1), lambda i: (i, 0)),
          pl.BlockSpec((1, _RB, 128), lambda i: (0, i, 0)),
          pl.BlockSpec((1, _RB, 128), lambda i: (1, i, 0)),
          pl.BlockSpec((_RB, 256), lambda i: (i, 0)),
          pl.BlockSpec((256, 512), lambda i: (0, 0)),
          pl.BlockSpec((1, 512), lambda i: (0, 0)),
          pl.BlockSpec((512, 128), lambda i: (0, 0)),
      ],
      out_specs=[
          pl.BlockSpec((_RB, 128), lambda i: (i, 0)),
          pl.BlockSpec((_RB, 128), lambda i: (i, 0)),
      ],
      out_shape=[
          jax.ShapeDtypeStruct((N, 128), jnp.float32),
          jax.ShapeDtypeStruct((N, 128), jnp.float32),
      ],
  )(d0, d1, p1, p1, x, w1, b1r, w2)


def _out_body(d0_ref, d1_ref, p2a_ref, p2b_ref, z_ref, b2_ref, o_ref):
  dis = _dis(d0_ref[...], d1_ref[...])
  dis2 = dis * dis
  p2 = p2a_ref[0] + p2b_ref[0]
  u = p2 * dis + z_ref[...] * dis2 + b2_ref[...]
  m = jnp.max(u, axis=1, keepdims=True)
  t = u - m
  lse = jnp.log(jnp.sum(jnp.exp(t), axis=1, keepdims=True))
  o_ref[...] = t - lse


def _tc_out(d0, d1, p2, z, b2r):
  return pl.pallas_call(
      _out_body,
      grid=(N // _RB,),
      in_specs=[
          pl.BlockSpec((_RB, 1), lambda i: (i, 0)),
          pl.BlockSpec((_RB, 1), lambda i: (i, 0)),
          pl.BlockSpec((1, _RB, 128), lambda i: (0, i, 0)),
          pl.BlockSpec((1, _RB, 128), lambda i: (1, i, 0)),
          pl.BlockSpec((_RB, 128), lambda i: (i, 0)),
          pl.BlockSpec((1, 128), lambda i: (0, 0)),
      ],
      out_specs=pl.BlockSpec((_RB, 128), lambda i: (i, 0)),
      out_shape=jax.ShapeDtypeStruct((N, 128), jnp.float32),
  )(d0, d1, p2, p2, z, b2r)


def kernel(x, edge_index, W1, b1, W2, b2):
  e3 = edge_index.reshape(2, E // 128, 128)
  parts, src2, dst2 = _sc_degree(e3)
  d0 = parts[0, :N].reshape(N, 1)
  d1 = parts[1, :N].reshape(N, 1)

  g1 = _tc_g1(d0, d1, x)
  p1 = _sc_propagate(g1, src2, dst2, True)
  z, g2 = _tc_mm(d0, d1, p1, x, W1,
                 b1.reshape(1, 512), W2)
  p2 = _sc_propagate(g2, src2, dst2, False)
  return _tc_out(d0, d1, p2, z, b2.reshape(1, 128))
